# baseline (speedup 1.0000x reference)
import dataclasses
import functools
import math

import jax
import jax.numpy as jnp
from jax import lax
from jax.experimental import pallas as pl
from jax.experimental.pallas import tpu as pltpu
from jax.experimental.pallas import tpu_sc as plsc

N = 10000
E = 160000
H = 128

_NC = 2
_NS = 16
_NW = _NC * _NS
_C = 128
_EROWS = 1280
_EPAD = _EROWS * _C
_RPW = _EROWS // _NW
_NP = 10240

_BE = 2048
_BN = 1000

_sc_mesh = plsc.VectorSubcoreMesh(core_axis_name="c", subcore_axis_name="s")

_sc_cp = pltpu.CompilerParams()
if "needs_layout_passes" in pltpu.CompilerParams.__dataclass_fields__:
    _sc_cp = dataclasses.replace(_sc_cp, needs_layout_passes=False)


def _sc_gather(h2b, x4, rowg, colg):
    nrows = rowg.shape[0] // _C
    rpw = nrows // _NW
    npad = nrows * _C

    @functools.partial(
        pl.kernel, mesh=_sc_mesh,
        out_type=[jax.ShapeDtypeStruct((npad, H), jnp.float32),
                  jax.ShapeDtypeStruct((npad, H), jnp.float32),
                  jax.ShapeDtypeStruct((8, npad), jnp.float32)],
        scratch_types=[pltpu.VMEM((rpw, _C), jnp.int32),
                       pltpu.VMEM((rpw, _C), jnp.int32),
                       pltpu.VMEM((_C, H), jnp.float32),
                       pltpu.VMEM((_C, H), jnp.float32),
                       pltpu.VMEM((_C, H), jnp.float32),
                       pltpu.VMEM((_C, H), jnp.float32),
                       pltpu.VMEM((4 * N,), jnp.float32),
                       pltpu.VMEM((8, _C), jnp.float32),
                       pltpu.VMEM((8, _C), jnp.float32),
                       pltpu.SemaphoreType.DMA,
                       pltpu.SemaphoreType.DMA],
        compiler_params=_sc_cp,
    )
    def k(h_hbm, x_hbm, ri_hbm, ci_hbm, hr_hbm, hc_hbm, rel_hbm,
          ribuf, cibuf, hbuf0, cbuf0, hbuf1, cbuf1, x4v, relbuf0, relbuf1,
          semg, semw):
        wid = lax.axis_index("s") * _NC + lax.axis_index("c")
        base = wid * rpw
        idma = []
        for j in range(rpw):
            idma.append(pltpu.async_copy(
                ri_hbm.at[pl.ds((base + j) * _C, _C)], ribuf.at[j], semg))
            idma.append(pltpu.async_copy(
                ci_hbm.at[pl.ds((base + j) * _C, _C)], cibuf.at[j], semg))
        pltpu.sync_copy(x_hbm, x4v)
        for d in idma:
            d.wait()
        zero16 = jnp.zeros((16,), jnp.float32)
        for rb in (relbuf0, relbuf1):
            for r in range(4, 8):
                for kk in range(8):
                    rb[r, pl.ds(kk * 16, 16)] = zero16

        def relcompute(j, rb):
            for kk in range(8):
                sl = pl.ds(kk * 16, 16)
                ir = ribuf[j, sl] * 4
                ic = cibuf[j, sl] * 4
                d2 = zero16
                for comp in range(3):
                    cidx = jnp.full((16,), comp, jnp.int32)
                    vr = plsc.load_gather(x4v, [ir + cidx])
                    vc = plsc.load_gather(x4v, [ic + cidx])
                    rr = vr - vc
                    rb[comp, sl] = rr
                    d2 = d2 + rr * rr
                rb[3, sl] = d2

        def drain_writes():
            pltpu.make_async_copy(hr_hbm.at[pl.ds(0, _C)], hbuf0,
                                  semw).wait()
            pltpu.make_async_copy(hc_hbm.at[pl.ds(0, _C)], cbuf0,
                                  semw).wait()
            pltpu.make_async_copy(rel_hbm.at[:, pl.ds(0, _C)], relbuf0,
                                  semw).wait()

        @pl.loop(0, rpw // 2)
        def _(jp):
            j0 = jp * 2

            @pl.when(jp > 0)
            def _():
                drain_writes()
                drain_writes()

            g0a = pltpu.async_copy(h_hbm.at[ribuf.at[j0]], hbuf0, semg)
            g0b = pltpu.async_copy(h_hbm.at[cibuf.at[j0]], cbuf0, semg)
            g1a = pltpu.async_copy(h_hbm.at[ribuf.at[j0 + 1]], hbuf1, semg)
            g1b = pltpu.async_copy(h_hbm.at[cibuf.at[j0 + 1]], cbuf1, semg)
            relcompute(j0, relbuf0)
            e0 = (base + j0) * _C
            g0a.wait()
            g0b.wait()
            pltpu.async_copy(hbuf0, hr_hbm.at[pl.ds(e0, _C)], semw)
            pltpu.async_copy(cbuf0, hc_hbm.at[pl.ds(e0, _C)], semw)
            pltpu.async_copy(relbuf0, rel_hbm.at[:, pl.ds(e0, _C)], semw)
            relcompute(j0 + 1, relbuf1)
            g1a.wait()
            g1b.wait()
            pltpu.async_copy(hbuf1, hr_hbm.at[pl.ds(e0 + _C, _C)], semw)
            pltpu.async_copy(cbuf1, hc_hbm.at[pl.ds(e0 + _C, _C)], semw)
            pltpu.async_copy(relbuf1, rel_hbm.at[:, pl.ds(e0 + _C, _C)],
                             semw)

        drain_writes()
        drain_writes()

    return k(h2b, x4, rowg, colg)


def _sc_scatter(msg, xmsg, rows, zm, zx):
    rps = _NP // _NS
    rpc = rows.shape[0] // _C // _NC
    rps_e = rpc // _NS

    nx = _NP * 4
    nxs = nx // _NS

    @functools.partial(
        pl.kernel, mesh=_sc_mesh,
        out_type=[jax.ShapeDtypeStruct((_NC, _NP, H), jnp.float32),
                  jax.ShapeDtypeStruct((_NC, nx), jnp.float32)],
        scratch_types=[pltpu.VMEM((rps_e, _C), jnp.int32),
                       pltpu.VMEM((_C, H), jnp.float32),
                       pltpu.VMEM((_C, H), jnp.float32),
                       pltpu.VMEM((4, _C), jnp.float32),
                       pltpu.VMEM((4, _C), jnp.float32),
                       pltpu.VMEM((4, _C), jnp.int32),
                       pltpu.VMEM_SHARED((_NP, H), jnp.float32),
                       pltpu.VMEM_SHARED((nx,), jnp.float32),
                       pltpu.SemaphoreType.DMA],
        compiler_params=_sc_cp,
    )
    def k(m_hbm, xm_hbm, ri_hbm, zm_hbm, zx_hbm, mo_hbm, xo_hbm,
          ibuf, mbuf0, mbuf1, xbuf0, xbuf1, ixbuf, macc, xacc, sem):
        c = lax.axis_index("c")
        s = lax.axis_index("s")
        pltpu.sync_copy(zm_hbm, macc.at[pl.ds(s * rps, rps)])
        pltpu.sync_copy(zx_hbm, xacc.at[pl.ds(s * nxs, nxs)])
        plsc.subcore_barrier()
        base = c * rpc + s * rps_e
        idma = []
        for j in range(rps_e):
            idma.append(pltpu.async_copy(
                ri_hbm.at[pl.ds((base + j) * _C, _C)], ibuf.at[j], sem))
        for d in idma:
            d.wait()

        def xscat(j, xbuf):
            for r in range(3):
                for g in range(8):
                    sl = pl.ds(g * 16, 16)
                    ixbuf[r, sl] = ibuf[j, sl] * 4 + r
            for r in range(3):
                pltpu.sync_copy(xbuf.at[r], xacc.at[ixbuf.at[r]],
                                add=True)

        @pl.loop(0, rps_e // 2)
        def _(jp):
            j0 = jp * 2
            e0 = (base + j0) * _C
            d0 = pltpu.async_copy(m_hbm.at[pl.ds(e0, _C)], mbuf0, sem)
            dx0 = pltpu.async_copy(xm_hbm.at[:, pl.ds(e0, _C)], xbuf0, sem)
            d1 = pltpu.async_copy(m_hbm.at[pl.ds(e0 + _C, _C)], mbuf1, sem)
            dx1 = pltpu.async_copy(xm_hbm.at[:, pl.ds(e0 + _C, _C)], xbuf1,
                                   sem)
            d0.wait()
            dx0.wait()
            pltpu.sync_copy(mbuf0, macc.at[ibuf.at[j0]], add=True)
            xscat(j0, xbuf0)
            d1.wait()
            dx1.wait()
            pltpu.sync_copy(mbuf1, macc.at[ibuf.at[j0 + 1]], add=True)
            xscat(j0 + 1, xbuf1)

        plsc.subcore_barrier()
        pltpu.sync_copy(macc.at[pl.ds(s * rps, rps)],
                        mo_hbm.at[c, pl.ds(s * rps, rps)])
        pltpu.sync_copy(xacc.at[pl.ds(s * nxs, nxs)],
                        xo_hbm.at[c, pl.ds(s * nxs, nxs)])

    return k(msg, xmsg, rows, zm, zx)


def _silu(v):
    return v * jax.nn.sigmoid(v)


def _edge_body(hr, hc, rel8, eye8, msk8, w1cat, w1d, b1, w2, b2,
               aw, ab, cw1, cb1, cw2, msg, xout):
    bf = jnp.bfloat16
    f32 = jnp.float32
    r8 = lax.dot_general(rel8[...], eye8[...], (((0,), (0,)), ((), ())),
                         preferred_element_type=f32)
    distb = jnp.sqrt(r8[:, 3:4]).astype(bf)
    hh = jnp.concatenate([hr[...].astype(bf), hc[...].astype(bf)], axis=1)
    t1 = (jnp.dot(hh, w1cat[...], preferred_element_type=f32).astype(bf)
          + distb * w1d[...] + b1[...])
    t1 = _silu(t1)
    m = jnp.dot(t1, w2[...], preferred_element_type=f32).astype(bf) + b2[...]
    m = _silu(m)
    att = jax.nn.sigmoid(jnp.dot(m, aw[...], preferred_element_type=f32)
                         + ab[...])
    msg[...] = att * m.astype(f32)
    c1 = _silu(jnp.dot(m, cw1[...], preferred_element_type=f32).astype(bf)
               + cb1[...])
    cwT = lax.dot_general(cw2[...], c1, (((1,), (1,)), ((), ())),
                          preferred_element_type=f32)
    distT = jnp.sqrt(rel8[3:4, :])
    xout[...] = (cwT * rel8[0:4, :] / (distT + 1e-8)) * msk8[...]


def _edge_block_call(hr, hc, rel8, eye8, msk8, wts):
    n_e = hr.shape[0]
    grid = n_e // _BE
    full = lambda s: pl.BlockSpec(s, lambda i: (0,) * len(s))
    eb = lambda d: pl.BlockSpec((_BE, d), lambda i: (i, 0))
    return pl.pallas_call(
        _edge_body,
        grid=(grid,),
        in_specs=[eb(H), eb(H), pl.BlockSpec((8, _BE), lambda i: (0, i)),
                  full((8, 8)), full((4, 1)),
                  full((2 * H, H)), full((1, H)), full((1, H)),
                  full((H, H)), full((1, H)), full((H, 1)), full((1, 1)),
                  full((H, H)), full((1, H)), full((1, H))],
        out_specs=[eb(H), pl.BlockSpec((4, _BE), lambda i: (0, i))],
        out_shape=[jax.ShapeDtypeStruct((n_e, H), jnp.float32),
                   jax.ShapeDtypeStruct((4, n_e), jnp.float32)],
        compiler_params=pltpu.CompilerParams(
            dimension_semantics=("parallel",)),
    )(hr, hc, rel8, eye8, msk8, *wts)


def _node_body(ln, nmo, h, *refs):
    ms = refs[:2 * nmo]
    w1a, w1b, b1, w2, b2, g, bv, out = refs[2 * nmo:]
    mi = ms[0][0]
    for mr in ms[1:]:
        mi = mi + mr[0]
    t = (jnp.dot(h[...], w1a[...], preferred_element_type=jnp.float32)
         + jnp.dot(mi, w1b[...], preferred_element_type=jnp.float32)
         + b1[...])
    t = _silu(t)
    hn = jnp.dot(t, w2[...], preferred_element_type=jnp.float32) + b2[...]
    hnew = h[...] + hn
    if ln:
        mu = jnp.mean(hnew, axis=-1, keepdims=True)
        va = jnp.mean((hnew - mu) ** 2, axis=-1, keepdims=True)
        hnew = (hnew - mu) / jnp.sqrt(va + 1e-5) * g[...] + bv[...]
    out[...] = hnew


def _node_call(ln, h, mos, w1a, w1b, b1, w2, b2, g, bv):
    grid = N // _BN
    full = lambda s: pl.BlockSpec(s, lambda i: (0,) * len(s))
    nb = pl.BlockSpec((_BN, H), lambda i: (i, 0))
    m0 = pl.BlockSpec((1, _BN, H), lambda i: (0, i, 0))
    m1 = pl.BlockSpec((1, _BN, H), lambda i: (1, i, 0))
    mspecs = []
    margs = []
    for mo in mos:
        mspecs += [m0, m1]
        margs += [mo, mo]
    return pl.pallas_call(
        functools.partial(_node_body, ln, len(mos)),
        grid=(grid,),
        in_specs=[nb] + mspecs +
                 [full((H, H)), full((H, H)), full((1, H)),
                  full((H, H)), full((1, H)), full((1, H)), full((1, H))],
        out_specs=nb,
        out_shape=jax.ShapeDtypeStruct((N, H), jnp.float32),
        compiler_params=pltpu.CompilerParams(
            dimension_semantics=("parallel",)),
    )(h, *margs, w1a, w1b, b1, w2, b2, g, bv)


def _precompute(t, af, dc, cc, p):
    silu = jax.nn.silu
    half = H // 2
    freqs = jnp.exp(jnp.arange(half, dtype=jnp.float32)
                    * (-(math.log(10000.0) / (half - 1))))
    te = t.astype(jnp.float32)[:, None] * freqs[None, :]
    temb = jnp.concatenate([jnp.sin(te), jnp.cos(te)], axis=-1)
    a = silu(af @ p['ce_aW1'] + p['ce_ab1']) @ p['ce_aW2'] + p['ce_ab2']
    a_emb = a.mean(axis=0, keepdims=True)
    d = silu(dc @ p['ce_dW1'] + p['ce_db1']) @ p['ce_dW2'] + p['ce_db2']
    d_emb = d.mean(axis=0, keepdims=True)
    c = silu(cc @ p['ce_cW1'] + p['ce_cb1']) @ p['ce_cW2'] + p['ce_cb2']
    c_emb = c.mean(axis=0, keepdims=True)
    comb = jnp.concatenate([a_emb, d_emb, c_emb], axis=-1)
    z = comb @ p['ce_fW1'] + p['ce_fb1']
    mu = z.mean(axis=-1, keepdims=True)
    va = ((z - mu) ** 2).mean(axis=-1, keepdims=True)
    z = (z - mu) / jnp.sqrt(va + 1e-5) * p['ce_fg'] + p['ce_fbe']
    cond = silu(z) @ p['ce_fW2'] + p['ce_fb2']
    tproj = temb @ p['tpW'] + p['tpb']
    cproj = cond @ p['cpW'] + p['cpb']
    return tproj + cproj


def kernel(h, x, edge_index, t, anchor_features, distance_constraints,
           coordination_constraints, params):
    p = params
    npad = _EPAD - E
    gpad = (jnp.arange(npad, dtype=jnp.int32) * 37) % N
    rowg = jnp.concatenate([edge_index[0], gpad])
    colg = jnp.concatenate([edge_index[1], gpad])
    spad = N + (jnp.arange(npad, dtype=jnp.int32) % (_NP - N))
    rows = jnp.concatenate([edge_index[0], spad])
    zm = jnp.zeros((_NP // _NS, H), jnp.float32)
    zx = jnp.zeros((_NP * 4 // _NS,), jnp.float32)
    eye8 = jnp.eye(8, dtype=jnp.float32)
    msk8 = jnp.array([[1.], [1.], [1.], [0.]], jnp.float32)

    h = h + _precompute(t, anchor_features, distance_constraints,
                        coordination_constraints, p)
    nchunk = 4
    crow = _EPAD // nchunk
    for i in range(2):
        x4 = jnp.pad(x, ((0, 0), (0, 1))).reshape(-1)
        bf = jnp.bfloat16
        wts = (p['eW1'][i, :2 * H].astype(bf),
               p['eW1'][i, 2 * H:].astype(bf), p['eb1'][i][None].astype(bf),
               p['eW2'][i].astype(bf), p['eb2'][i][None].astype(bf),
               p['aW'][i].astype(bf), p['ab'][i][None],
               p['cW1'][i].astype(bf), p['cb1'][i][None].astype(bf),
               p['cW2'][i].T.astype(bf))
        parts = []
        for ci in range(nchunk):
            hs = slice(ci * crow, (ci + 1) * crow)
            hr, hc, rel8 = _sc_gather(h, x4, rowg[hs], colg[hs])
            msg, xmsg = _edge_block_call(hr, hc, rel8, eye8, msk8, wts)
            parts.append(_sc_scatter(msg, xmsg, rows[hs], zm, zx))
        h = _node_call(i == 1, h, [mo for mo, _ in parts],
                       p['nW1'][i, :H], p['nW1'][i, H:], p['nb1'][i][None],
                       p['nW2'][i], p['nb2'][i][None],
                       p['ln_g'][None], p['ln_b'][None])
        xacc = parts[0][1][0] + parts[0][1][1]
        for _, xo in parts[1:]:
            xacc = xacc + xo[0] + xo[1]
        x = x + xacc.reshape(_NP, 4)[:N, :3]
    return h, x


# speedup vs baseline: 1.0675x; 1.0675x over previous
import dataclasses
import functools
import math

import jax
import jax.numpy as jnp
from jax import lax
from jax.experimental import pallas as pl
from jax.experimental.pallas import tpu as pltpu
from jax.experimental.pallas import tpu_sc as plsc

N = 10000
E = 160000
H = 128

_NC = 2
_NS = 16
_NW = _NC * _NS
_C = 128
_EROWS = 1280
_EPAD = _EROWS * _C
_RPW = _EROWS // _NW
_NP = 10240

_BE = 1024
_BN = 1000

_sc_mesh = plsc.VectorSubcoreMesh(core_axis_name="c", subcore_axis_name="s")

_sc_cp = pltpu.CompilerParams()
if "needs_layout_passes" in pltpu.CompilerParams.__dataclass_fields__:
    _sc_cp = dataclasses.replace(_sc_cp, needs_layout_passes=False)


def _sc_gather(h2b, x4, rowg, colg):
    nrows = rowg.shape[0] // _C
    rpw = nrows // _NW
    npad = nrows * _C

    @functools.partial(
        pl.kernel, mesh=_sc_mesh,
        out_type=[jax.ShapeDtypeStruct((npad, H), jnp.float32),
                  jax.ShapeDtypeStruct((npad, H), jnp.float32),
                  jax.ShapeDtypeStruct((8, npad), jnp.float32)],
        scratch_types=[pltpu.VMEM((rpw, _C), jnp.int32),
                       pltpu.VMEM((rpw, _C), jnp.int32),
                       pltpu.VMEM((_C, H), jnp.float32),
                       pltpu.VMEM((_C, H), jnp.float32),
                       pltpu.VMEM((_C, H), jnp.float32),
                       pltpu.VMEM((_C, H), jnp.float32),
                       pltpu.VMEM((4 * N,), jnp.float32),
                       pltpu.VMEM((8, _C), jnp.float32),
                       pltpu.VMEM((8, _C), jnp.float32),
                       pltpu.SemaphoreType.DMA,
                       pltpu.SemaphoreType.DMA],
        compiler_params=_sc_cp,
    )
    def k(h_hbm, x_hbm, ri_hbm, ci_hbm, hr_hbm, hc_hbm, rel_hbm,
          ribuf, cibuf, hbuf0, cbuf0, hbuf1, cbuf1, x4v, relbuf0, relbuf1,
          semg, semw):
        wid = lax.axis_index("s") * _NC + lax.axis_index("c")
        base = wid * rpw
        idma = []
        for j in range(rpw):
            idma.append(pltpu.async_copy(
                ri_hbm.at[pl.ds((base + j) * _C, _C)], ribuf.at[j], semg))
            idma.append(pltpu.async_copy(
                ci_hbm.at[pl.ds((base + j) * _C, _C)], cibuf.at[j], semg))
        pltpu.sync_copy(x_hbm, x4v)
        for d in idma:
            d.wait()
        zero16 = jnp.zeros((16,), jnp.float32)
        for rb in (relbuf0, relbuf1):
            for r in range(4, 8):
                for kk in range(8):
                    rb[r, pl.ds(kk * 16, 16)] = zero16

        def relcompute(j, rb):
            for kk in range(8):
                sl = pl.ds(kk * 16, 16)
                ir = ribuf[j, sl] * 4
                ic = cibuf[j, sl] * 4
                d2 = zero16
                for comp in range(3):
                    cidx = jnp.full((16,), comp, jnp.int32)
                    vr = plsc.load_gather(x4v, [ir + cidx])
                    vc = plsc.load_gather(x4v, [ic + cidx])
                    rr = vr - vc
                    rb[comp, sl] = rr
                    d2 = d2 + rr * rr
                rb[3, sl] = d2

        def drain_writes():
            pltpu.make_async_copy(hr_hbm.at[pl.ds(0, _C)], hbuf0,
                                  semw).wait()
            pltpu.make_async_copy(hc_hbm.at[pl.ds(0, _C)], cbuf0,
                                  semw).wait()
            pltpu.make_async_copy(rel_hbm.at[:, pl.ds(0, _C)], relbuf0,
                                  semw).wait()

        @pl.loop(0, rpw // 2)
        def _(jp):
            j0 = jp * 2

            @pl.when(jp > 0)
            def _():
                drain_writes()
                drain_writes()

            g0a = pltpu.async_copy(h_hbm.at[ribuf.at[j0]], hbuf0, semg)
            g0b = pltpu.async_copy(h_hbm.at[cibuf.at[j0]], cbuf0, semg)
            g1a = pltpu.async_copy(h_hbm.at[ribuf.at[j0 + 1]], hbuf1, semg)
            g1b = pltpu.async_copy(h_hbm.at[cibuf.at[j0 + 1]], cbuf1, semg)
            relcompute(j0, relbuf0)
            e0 = (base + j0) * _C
            g0a.wait()
            g0b.wait()
            pltpu.async_copy(hbuf0, hr_hbm.at[pl.ds(e0, _C)], semw)
            pltpu.async_copy(cbuf0, hc_hbm.at[pl.ds(e0, _C)], semw)
            pltpu.async_copy(relbuf0, rel_hbm.at[:, pl.ds(e0, _C)], semw)
            relcompute(j0 + 1, relbuf1)
            g1a.wait()
            g1b.wait()
            pltpu.async_copy(hbuf1, hr_hbm.at[pl.ds(e0 + _C, _C)], semw)
            pltpu.async_copy(cbuf1, hc_hbm.at[pl.ds(e0 + _C, _C)], semw)
            pltpu.async_copy(relbuf1, rel_hbm.at[:, pl.ds(e0 + _C, _C)],
                             semw)

        drain_writes()
        drain_writes()

    return k(h2b, x4, rowg, colg)


def _sc_scatter(msg, xmsg, rows):
    rps = _NP // _NS
    rpc = rows.shape[0] // _C // _NC
    rps_e = rpc // _NS

    nx = _NP * 4
    nxs = nx // _NS

    @functools.partial(
        pl.kernel, mesh=_sc_mesh,
        out_type=[jax.ShapeDtypeStruct((_NC, _NP, H), jnp.float32),
                  jax.ShapeDtypeStruct((_NC, nx), jnp.float32)],
        scratch_types=[pltpu.VMEM((rps_e, _C), jnp.int32),
                       pltpu.VMEM((_C, H), jnp.float32),
                       pltpu.VMEM((_C, H), jnp.float32),
                       pltpu.VMEM((4, _C), jnp.float32),
                       pltpu.VMEM((4, _C), jnp.float32),
                       pltpu.VMEM((4, _C), jnp.int32),
                       pltpu.VMEM((nxs,), jnp.float32),
                       pltpu.VMEM_SHARED((_NP, H), jnp.float32),
                       pltpu.VMEM_SHARED((nx,), jnp.float32),
                       pltpu.SemaphoreType.DMA],
        compiler_params=_sc_cp,
    )
    def k(m_hbm, xm_hbm, ri_hbm, mo_hbm, xo_hbm,
          ibuf, mbuf0, mbuf1, xbuf0, xbuf1, ixbuf, zbx, macc, xacc,
          sem):
        c = lax.axis_index("c")
        s = lax.axis_index("s")
        zero16 = jnp.zeros((16,), jnp.float32)

        @pl.loop(0, _C)
        def _(r):
            for kk in range(H // 16):
                mbuf0[r, pl.ds(kk * 16, 16)] = zero16

        @pl.loop(0, nxs // 16)
        def _(r):
            zbx[pl.ds(r * 16, 16)] = zero16

        zdma = [pltpu.async_copy(zbx, xacc.at[pl.ds(s * nxs, nxs)], sem)]
        for kk in range(rps // _C):
            zdma.append(pltpu.async_copy(
                mbuf0, macc.at[pl.ds(s * rps + kk * _C, _C)], sem))
        for d in zdma:
            d.wait()
        plsc.subcore_barrier()
        base = c * rpc + s * rps_e
        idma = []
        for j in range(rps_e):
            idma.append(pltpu.async_copy(
                ri_hbm.at[pl.ds((base + j) * _C, _C)], ibuf.at[j], sem))
        for d in idma:
            d.wait()

        def xscat(j, xbuf):
            for r in range(3):
                for g in range(8):
                    sl = pl.ds(g * 16, 16)
                    ixbuf[r, sl] = ibuf[j, sl] * 4 + r
            for r in range(3):
                pltpu.sync_copy(xbuf.at[r], xacc.at[ixbuf.at[r]],
                                add=True)

        @pl.loop(0, rps_e // 2)
        def _(jp):
            j0 = jp * 2
            e0 = (base + j0) * _C
            d0 = pltpu.async_copy(m_hbm.at[pl.ds(e0, _C)], mbuf0, sem)
            dx0 = pltpu.async_copy(xm_hbm.at[:, pl.ds(e0, _C)], xbuf0, sem)
            d1 = pltpu.async_copy(m_hbm.at[pl.ds(e0 + _C, _C)], mbuf1, sem)
            dx1 = pltpu.async_copy(xm_hbm.at[:, pl.ds(e0 + _C, _C)], xbuf1,
                                   sem)
            d0.wait()
            dx0.wait()
            pltpu.sync_copy(mbuf0, macc.at[ibuf.at[j0]], add=True)
            xscat(j0, xbuf0)
            d1.wait()
            dx1.wait()
            pltpu.sync_copy(mbuf1, macc.at[ibuf.at[j0 + 1]], add=True)
            xscat(j0 + 1, xbuf1)

        plsc.subcore_barrier()
        pltpu.sync_copy(macc.at[pl.ds(s * rps, rps)],
                        mo_hbm.at[c, pl.ds(s * rps, rps)])
        pltpu.sync_copy(xacc.at[pl.ds(s * nxs, nxs)],
                        xo_hbm.at[c, pl.ds(s * nxs, nxs)])

    return k(msg, xmsg, rows)


def _silu(v):
    return v * jax.nn.sigmoid(v)


def _edge_body(hr, hc, rel8, eye8, msk8, w1cat, w1d, b1, w2, b2,
               aw, ab, cw1, cb1, cw2, msg, xout):
    bf = jnp.bfloat16
    f32 = jnp.float32
    r8 = lax.dot_general(rel8[...], eye8[...], (((0,), (0,)), ((), ())),
                         preferred_element_type=f32)
    distb = jnp.sqrt(r8[:, 3:4]).astype(bf)
    hh = jnp.concatenate([hr[...].astype(bf), hc[...].astype(bf)], axis=1)
    t1 = (jnp.dot(hh, w1cat[...], preferred_element_type=f32).astype(bf)
          + distb * w1d[...] + b1[...])
    t1 = _silu(t1)
    m = jnp.dot(t1, w2[...], preferred_element_type=f32).astype(bf) + b2[...]
    m = _silu(m)
    att = jax.nn.sigmoid(jnp.dot(m, aw[...], preferred_element_type=f32)
                         + ab[...])
    msg[...] = att * m.astype(f32)
    c1 = _silu(jnp.dot(m, cw1[...], preferred_element_type=f32).astype(bf)
               + cb1[...])
    cwT = lax.dot_general(cw2[...], c1, (((1,), (1,)), ((), ())),
                          preferred_element_type=f32)
    distT = jnp.sqrt(rel8[3:4, :])
    xout[...] = (cwT * rel8[0:4, :] / (distT + 1e-8)) * msk8[...]


def _edge_block_call(hr, hc, rel8, eye8, msk8, wts):
    n_e = hr.shape[0]
    grid = n_e // _BE
    full = lambda s: pl.BlockSpec(s, lambda i: (0,) * len(s))
    eb = lambda d: pl.BlockSpec((_BE, d), lambda i: (i, 0))
    return pl.pallas_call(
        _edge_body,
        grid=(grid,),
        in_specs=[eb(H), eb(H), pl.BlockSpec((8, _BE), lambda i: (0, i)),
                  full((8, 8)), full((4, 1)),
                  full((2 * H, H)), full((1, H)), full((1, H)),
                  full((H, H)), full((1, H)), full((H, 1)), full((1, 1)),
                  full((H, H)), full((1, H)), full((1, H))],
        out_specs=[eb(H), pl.BlockSpec((4, _BE), lambda i: (0, i))],
        out_shape=[jax.ShapeDtypeStruct((n_e, H), jnp.float32),
                   jax.ShapeDtypeStruct((4, n_e), jnp.float32)],
        compiler_params=pltpu.CompilerParams(
            dimension_semantics=("parallel",)),
    )(hr, hc, rel8, eye8, msk8, *wts)


def _node_body(ln, nmo, h, *refs):
    ms = refs[:2 * nmo]
    w1a, w1b, b1, w2, b2, g, bv, out = refs[2 * nmo:]
    mi = ms[0][0]
    for mr in ms[1:]:
        mi = mi + mr[0]
    t = (jnp.dot(h[...], w1a[...], preferred_element_type=jnp.float32)
         + jnp.dot(mi, w1b[...], preferred_element_type=jnp.float32)
         + b1[...])
    t = _silu(t)
    hn = jnp.dot(t, w2[...], preferred_element_type=jnp.float32) + b2[...]
    hnew = h[...] + hn
    if ln:
        mu = jnp.mean(hnew, axis=-1, keepdims=True)
        va = jnp.mean((hnew - mu) ** 2, axis=-1, keepdims=True)
        hnew = (hnew - mu) / jnp.sqrt(va + 1e-5) * g[...] + bv[...]
    out[...] = hnew


def _node_call(ln, h, mos, w1a, w1b, b1, w2, b2, g, bv):
    grid = N // _BN
    full = lambda s: pl.BlockSpec(s, lambda i: (0,) * len(s))
    nb = pl.BlockSpec((_BN, H), lambda i: (i, 0))
    m0 = pl.BlockSpec((1, _BN, H), lambda i: (0, i, 0))
    m1 = pl.BlockSpec((1, _BN, H), lambda i: (1, i, 0))
    mspecs = []
    margs = []
    for mo in mos:
        mspecs += [m0, m1]
        margs += [mo, mo]
    return pl.pallas_call(
        functools.partial(_node_body, ln, len(mos)),
        grid=(grid,),
        in_specs=[nb] + mspecs +
                 [full((H, H)), full((H, H)), full((1, H)),
                  full((H, H)), full((1, H)), full((1, H)), full((1, H))],
        out_specs=nb,
        out_shape=jax.ShapeDtypeStruct((N, H), jnp.float32),
        compiler_params=pltpu.CompilerParams(
            dimension_semantics=("parallel",)),
    )(h, *margs, w1a, w1b, b1, w2, b2, g, bv)


def _precompute(t, af, dc, cc, p):
    silu = jax.nn.silu
    half = H // 2
    freqs = jnp.exp(jnp.arange(half, dtype=jnp.float32)
                    * (-(math.log(10000.0) / (half - 1))))
    te = t.astype(jnp.float32)[:, None] * freqs[None, :]
    temb = jnp.concatenate([jnp.sin(te), jnp.cos(te)], axis=-1)
    a = silu(af @ p['ce_aW1'] + p['ce_ab1']) @ p['ce_aW2'] + p['ce_ab2']
    a_emb = a.mean(axis=0, keepdims=True)
    d = silu(dc @ p['ce_dW1'] + p['ce_db1']) @ p['ce_dW2'] + p['ce_db2']
    d_emb = d.mean(axis=0, keepdims=True)
    c = silu(cc @ p['ce_cW1'] + p['ce_cb1']) @ p['ce_cW2'] + p['ce_cb2']
    c_emb = c.mean(axis=0, keepdims=True)
    comb = jnp.concatenate([a_emb, d_emb, c_emb], axis=-1)
    z = comb @ p['ce_fW1'] + p['ce_fb1']
    mu = z.mean(axis=-1, keepdims=True)
    va = ((z - mu) ** 2).mean(axis=-1, keepdims=True)
    z = (z - mu) / jnp.sqrt(va + 1e-5) * p['ce_fg'] + p['ce_fbe']
    cond = silu(z) @ p['ce_fW2'] + p['ce_fb2']
    tproj = temb @ p['tpW'] + p['tpb']
    cproj = cond @ p['cpW'] + p['cpb']
    return tproj + cproj


def kernel(h, x, edge_index, t, anchor_features, distance_constraints,
           coordination_constraints, params):
    p = params
    npad = _EPAD - E
    gpad = (jnp.arange(npad, dtype=jnp.int32) * 37) % N
    rowg = jnp.concatenate([edge_index[0], gpad])
    colg = jnp.concatenate([edge_index[1], gpad])
    spad = N + (jnp.arange(npad, dtype=jnp.int32) % (_NP - N))
    rows = jnp.concatenate([edge_index[0], spad])
    eye8 = jnp.eye(8, dtype=jnp.float32)
    msk8 = jnp.array([[1.], [1.], [1.], [0.]], jnp.float32)

    h = h + _precompute(t, anchor_features, distance_constraints,
                        coordination_constraints, p)
    nchunk = 4
    crow = _EPAD // nchunk
    for i in range(2):
        x4 = jnp.pad(x, ((0, 0), (0, 1))).reshape(-1)
        bf = jnp.bfloat16
        wts = (p['eW1'][i, :2 * H].astype(bf),
               p['eW1'][i, 2 * H:].astype(bf), p['eb1'][i][None].astype(bf),
               p['eW2'][i].astype(bf), p['eb2'][i][None].astype(bf),
               p['aW'][i].astype(bf), p['ab'][i][None],
               p['cW1'][i].astype(bf), p['cb1'][i][None].astype(bf),
               p['cW2'][i].T.astype(bf))
        parts = []
        for ci in range(nchunk):
            hs = slice(ci * crow, (ci + 1) * crow)
            hr, hc, rel8 = _sc_gather(h, x4, rowg[hs], colg[hs])
            msg, xmsg = _edge_block_call(hr, hc, rel8, eye8, msk8, wts)
            parts.append(_sc_scatter(msg, xmsg, rows[hs]))
        h = _node_call(i == 1, h, [mo for mo, _ in parts],
                       p['nW1'][i, :H], p['nW1'][i, H:], p['nb1'][i][None],
                       p['nW2'][i], p['nb2'][i][None],
                       p['ln_g'][None], p['ln_b'][None])
        xacc = parts[0][1][0] + parts[0][1][1]
        for _, xo in parts[1:]:
            xacc = xacc + xo[0] + xo[1]
        x = x + xacc.reshape(_NP, 4)[:N, :3]
    return h, x
